# Initial kernel scaffold; baseline (speedup 1.0000x reference)
import jax
import jax.numpy as jnp
from jax.experimental import pallas as pl


def kernel(x, edge_index, edge_attr, batch, nA, nB, system_size, params):
    raise NotImplementedError("write your pallas kernel here")


# speedup vs baseline: 2.4101x; 2.4101x over previous
import functools
import math

import jax
import jax.numpy as jnp
from jax import lax
from jax.experimental import pallas as pl
from jax.experimental.pallas import tpu as pltpu
from jax.experimental.pallas import tpu_sc as plsc

N = 10000
E = 160000
G = 16
H = 256
HEADS = 8
C = H // HEADS
NL = 8

EB = 2000
NB = 2000

_INTERPRET = False

_SC_CORES = 2
_SC_TILES = 16
_SC_WORKERS = _SC_CORES * _SC_TILES


def _rb(cols, blk):
    return pl.BlockSpec((blk, cols), lambda i: (i, 0))


def _wfull(shape):
    return pl.BlockSpec(shape, lambda i: (0,) * len(shape))


def _lnrow(y, g, be):
    m = jnp.mean(y, axis=1, keepdims=True)
    v = jnp.mean((y - m) ** 2, axis=1, keepdims=True)
    return (y - m) * lax.rsqrt(v + 1e-5) * g + be


def _silu(y):
    return y * jax.nn.sigmoid(y)


def _dot(a, b):
    return jnp.dot(a, b, preferred_element_type=jnp.float32)


def _enc_body(x_ref, wt_ref, b_ref, g_ref, be_ref, o_ref):
    y = _dot(x_ref[...], wt_ref[...]) + b_ref[...]
    o_ref[...] = _silu(_lnrow(y, g_ref[...], be_ref[...]))


@functools.lru_cache(maxsize=None)
def _enc_call(rows, kin, blk):
    return pl.pallas_call(
        _enc_body,
        grid=(rows // blk,),
        in_specs=[_rb(kin, blk), _wfull((kin, H)), _wfull((1, H)),
                  _wfull((1, H)), _wfull((1, H))],
        out_specs=_rb(H, blk),
        out_shape=jax.ShapeDtypeStruct((rows, H), jnp.float32),
        interpret=_INTERPRET,
    )


def _gine_edge_body(e_ref, hs_ref, wt_ref, b_ref, g_ref, be_ref,
                    eo_ref, msg_ref):
    y = _dot(e_ref[...], wt_ref[...]) + b_ref[...]
    e_new = _silu(_lnrow(y, g_ref[...], be_ref[...]))
    eo_ref[...] = e_new
    msg_ref[...] = jnp.maximum(hs_ref[...] + e_new, 0.0)


@functools.lru_cache(maxsize=None)
def _gine_edge_call():
    return pl.pallas_call(
        _gine_edge_body,
        grid=(E // EB,),
        in_specs=[_rb(H, EB), _rb(H, EB), _wfull((H, H)), _wfull((1, H)),
                  _wfull((1, H)), _wfull((1, H))],
        out_specs=[_rb(H, EB), _rb(H, EB)],
        out_shape=[jax.ShapeDtypeStruct((E, H), jnp.float32),
                   jax.ShapeDtypeStruct((E, H), jnp.float32)],
        interpret=_INTERPRET,
    )


def _gine_node_body(h_ref, agg_ref, w1t_ref, b1_ref, g1_ref, be1_ref,
                    w2t_ref, b2_ref, ng_ref, nbe_ref,
                    wpt_ref, bp_ref, gp_ref, bep_ref, o_ref):
    z = h_ref[...] + agg_ref[...]
    z = _dot(z, w1t_ref[...]) + b1_ref[...]
    z = _silu(_lnrow(z, g1_ref[...], be1_ref[...]))
    z = _dot(z, w2t_ref[...]) + b2_ref[...]
    h2 = h_ref[...] + _lnrow(z, ng_ref[...], nbe_ref[...])
    y = _dot(h2, wpt_ref[...]) + bp_ref[...]
    o_ref[...] = _silu(_lnrow(y, gp_ref[...], bep_ref[...]))


@functools.lru_cache(maxsize=None)
def _gine_node_call():
    return pl.pallas_call(
        _gine_node_body,
        grid=(N // NB,),
        in_specs=[_rb(H, NB), _rb(H, NB),
                  _wfull((H, H)), _wfull((1, H)), _wfull((1, H)),
                  _wfull((1, H)),
                  _wfull((H, H)), _wfull((1, H)),
                  _wfull((1, H)), _wfull((1, H)),
                  _wfull((H, H)), _wfull((1, H)), _wfull((1, H)),
                  _wfull((1, H))],
        out_specs=_rb(H, NB),
        out_shape=jax.ShapeDtypeStruct((N, H), jnp.float32),
        interpret=_INTERPRET,
    )


def _tf_edge_body(e_ref, wt_ref, b_ref, g_ref, be_ref, wet_ref,
                  eo_ref, ep_ref):
    y = _dot(e_ref[...], wt_ref[...]) + b_ref[...]
    e_new = _silu(_lnrow(y, g_ref[...], be_ref[...]))
    eo_ref[...] = e_new
    ep_ref[...] = _dot(e_new, wet_ref[...])


@functools.lru_cache(maxsize=None)
def _tf_edge_call():
    return pl.pallas_call(
        _tf_edge_body,
        grid=(E // EB,),
        in_specs=[_rb(H, EB), _wfull((H, H)), _wfull((1, H)),
                  _wfull((1, H)), _wfull((1, H)), _wfull((H, H))],
        out_specs=[_rb(H, EB), _rb(H, EB)],
        out_shape=[jax.ShapeDtypeStruct((E, H), jnp.float32),
                   jax.ShapeDtypeStruct((E, H), jnp.float32)],
        interpret=_INTERPRET,
    )


def _qkvs_body(h_ref, wt_ref, b_ref, o_ref):
    o_ref[...] = _dot(h_ref[...], wt_ref[...]) + b_ref[...]


@functools.lru_cache(maxsize=None)
def _qkvs_call():
    return pl.pallas_call(
        _qkvs_body,
        grid=(N // NB,),
        in_specs=[_rb(H, NB), _wfull((H, 4 * H)), _wfull((1, 4 * H))],
        out_specs=_rb(4 * H, NB),
        out_shape=jax.ShapeDtypeStruct((N, 4 * H), jnp.float32),
        interpret=_INTERPRET,
    )


def _logits_body(qd_ref, ks_ref, ep_ref, s_ref, lg_ref, m_ref):
    i = pl.program_id(0)
    kj = ks_ref[...] + ep_ref[...]
    prod = qd_ref[...] * kj
    lg = _dot(prod, s_ref[...]) * (1.0 / math.sqrt(float(C)))
    lg_ref[...] = lg
    mblk = jnp.max(lg, axis=0, keepdims=True)

    @pl.when(i == 0)
    def _():
        m_ref[...] = mblk

    @pl.when(i > 0)
    def _():
        m_ref[...] = jnp.maximum(m_ref[...], mblk)


@functools.lru_cache(maxsize=None)
def _logits_call():
    return pl.pallas_call(
        _logits_body,
        grid=(E // EB,),
        in_specs=[_rb(H, EB), _rb(H, EB), _rb(H, EB), _wfull((H, HEADS))],
        out_specs=[_rb(HEADS, EB), _wfull((1, HEADS))],
        out_shape=[jax.ShapeDtypeStruct((E, HEADS), jnp.float32),
                   jax.ShapeDtypeStruct((1, HEADS), jnp.float32)],
        interpret=_INTERPRET,
    )


def _attnmsg_body(lg_ref, m_ref, vs_ref, ep_ref, st_ref,
                  exv_ref, expad_ref):
    ex = jnp.exp(lg_ref[...] - m_ref[...])
    exb = _dot(ex, st_ref[...])
    exv_ref[...] = exb * (vs_ref[...] + ep_ref[...])
    expad_ref[...] = jnp.concatenate(
        [ex, jnp.zeros_like(ex)], axis=1)


@functools.lru_cache(maxsize=None)
def _attnmsg_call():
    return pl.pallas_call(
        _attnmsg_body,
        grid=(E // EB,),
        in_specs=[_rb(HEADS, EB), _wfull((1, HEADS)), _rb(H, EB),
                  _rb(H, EB), _wfull((HEADS, H))],
        out_specs=[_rb(H, EB), _rb(2 * HEADS, EB)],
        out_shape=[jax.ShapeDtypeStruct((E, H), jnp.float32),
                   jax.ShapeDtypeStruct((E, 2 * HEADS), jnp.float32)],
        interpret=_INTERPRET,
    )


def _tf_node_body(num_ref, den_ref, xr_ref, h_ref, sp_ref, wb_ref,
                  ng_ref, nbe_ref, o_ref):
    denb = _dot(den_ref[...], sp_ref[...])
    out = num_ref[...] / (denb + 1e-16)
    xr = xr_ref[...]
    cat = jnp.concatenate([out, xr, out - xr], axis=1)
    beta = jax.nn.sigmoid(_dot(cat, wb_ref[...]))
    h_new = beta * xr + (1.0 - beta) * out
    o_ref[...] = h_ref[...] + _lnrow(h_new, ng_ref[...], nbe_ref[...])


@functools.lru_cache(maxsize=None)
def _tf_node_call():
    return pl.pallas_call(
        _tf_node_body,
        grid=(N // NB,),
        in_specs=[_rb(H, NB), _rb(2 * HEADS, NB), _rb(H, NB), _rb(H, NB),
                  _wfull((2 * HEADS, H)), _wfull((3 * H, H)),
                  _wfull((1, H)), _wfull((1, H))],
        out_specs=_rb(H, NB),
        out_shape=jax.ShapeDtypeStruct((N, H), jnp.float32),
        interpret=_INTERPRET,
    )


def _s2s_tail_body(h_ref, bf_ref, oht_ref, nap_ref, nbp_ref, ssp_ref,
                   wih0_ref, whh0_ref, bih0_ref, bhh0_ref,
                   wih1_ref, whh1_ref, bih1_ref, bhh1_ref,
                   wro_ref, bro_ref, gro_ref, bero_ref,
                   wg1_ref, bg1_ref, gg_ref, beg_ref, wg2_ref, bg2_ref,
                   wf1_ref, bf1_ref, gf1_ref, bef1_ref,
                   wf2_ref, bf2_ref, gf2_ref, bef2_ref,
                   wf3_ref, bf3_ref, o_ref):
    hv = h_ref[...]
    bf = bf_ref[...]
    col = lax.broadcasted_iota(jnp.int32, (N, 128), 1).astype(jnp.float32)
    onehot = jnp.where(bf == col, 1.0, 0.0)
    oh16 = onehot[:, :G]
    oht = oht_ref[...]

    def s2s(wih, whh, bih, bhh):
        hs = jnp.zeros((G, H), jnp.float32)
        cs = jnp.zeros((G, H), jnp.float32)
        q_star = jnp.zeros((G, 2 * H), jnp.float32)
        for _ in range(4):
            gates = (_dot(q_star, wih) + bih + _dot(hs, whh) + bhh)
            ig = jax.nn.sigmoid(gates[:, 0:H])
            fg = jax.nn.sigmoid(gates[:, H:2 * H])
            gg_ = jnp.tanh(gates[:, 2 * H:3 * H])
            og = jax.nn.sigmoid(gates[:, 3 * H:4 * H])
            cs = fg * cs + ig * gg_
            hs = og * jnp.tanh(cs)
            qb = _dot(oh16, hs)
            e = jnp.sum(hv * qb, axis=1, keepdims=True)
            eb = e - 1e30 * (1.0 - onehot)
            m128 = jnp.max(eb, axis=0, keepdims=True)
            mb = jnp.sum(onehot * m128, axis=1, keepdims=True)
            ex = jnp.exp(e - mb)
            s128 = jnp.sum(onehot * ex, axis=0, keepdims=True)
            sb = jnp.sum(onehot * s128, axis=1, keepdims=True)
            a = ex / (sb + 1e-16)
            r = _dot(oht, a * hv)
            q_star = jnp.concatenate([hs, r], axis=1)
        return q_star

    q0 = s2s(wih0_ref[...], whh0_ref[...], bih0_ref[...], bhh0_ref[...])
    q1 = s2s(wih1_ref[...], whh1_ref[...], bih1_ref[...], bhh1_ref[...])
    h_r = jnp.concatenate([q0, q1], axis=1)
    h_r = _silu(_lnrow(_dot(h_r, wro_ref[...]) + bro_ref[...],
                       gro_ref[...], bero_ref[...]))
    ccol = lax.broadcasted_iota(jnp.int32, (G, 128), 1).astype(jnp.float32)
    nav = nap_ref[...] / (ssp_ref[...] + 1e-10)
    nbv = nbp_ref[...] / (ssp_ref[...] + 1e-10)
    gf = jnp.where(ccol == 0.0, nav, jnp.where(ccol == 1.0, nbv, 0.0))
    g1 = _silu(_lnrow(_dot(gf, wg1_ref[...]) + bg1_ref[...],
                      gg_ref[...], beg_ref[...]))
    gf_out = _dot(g1, wg2_ref[...]) + bg2_ref[...]
    comb = jnp.concatenate([h_r, gf_out], axis=1)
    z = _silu(_lnrow(_dot(comb, wf1_ref[...]) + bf1_ref[...],
                     gf1_ref[...], bef1_ref[...]))
    z = _silu(_lnrow(_dot(z, wf2_ref[...]) + bf2_ref[...],
                     gf2_ref[...], bef2_ref[...]))
    zz = _dot(z, wf3_ref[...]) + bf3_ref[...]
    o_ref[...] = jnp.maximum(zz, 0.0) + jnp.log(1.0 + jnp.exp(-jnp.abs(zz)))


@functools.lru_cache(maxsize=None)
def _s2s_tail_call():
    ws = [
        _wfull((N, H)), _wfull((N, 128)), _wfull((G, N)),
        _wfull((G, 128)), _wfull((G, 128)), _wfull((G, 128)),
        _wfull((2 * H, 4 * H)), _wfull((H, 4 * H)), _wfull((1, 4 * H)),
        _wfull((1, 4 * H)),
        _wfull((2 * H, 4 * H)), _wfull((H, 4 * H)), _wfull((1, 4 * H)),
        _wfull((1, 4 * H)),
        _wfull((4 * H, 2 * H)), _wfull((1, 2 * H)), _wfull((1, 2 * H)),
        _wfull((1, 2 * H)),
        _wfull((128, H)), _wfull((1, H)), _wfull((1, H)), _wfull((1, H)),
        _wfull((H, H)), _wfull((1, H)),
        _wfull((3 * H, H)), _wfull((1, H)), _wfull((1, H)), _wfull((1, H)),
        _wfull((H, 128)), _wfull((1, 128)), _wfull((1, 128)),
        _wfull((1, 128)),
        _wfull((128, 128)), _wfull((1, 128)),
    ]
    return pl.pallas_call(
        _s2s_tail_body,
        grid=(1,),
        in_specs=ws,
        out_specs=_wfull((G, 128)),
        out_shape=jax.ShapeDtypeStruct((G, 128), jnp.float32),
        interpret=_INTERPRET,
    )


@functools.lru_cache(maxsize=None)
def _gather_call(t_rows, cols):
    per = E // _SC_WORKERS
    ch = 40
    iters = per // ch
    mesh = plsc.VectorSubcoreMesh(core_axis_name="c", subcore_axis_name="s")

    @functools.partial(
        pl.kernel,
        mesh=mesh,
        compiler_params=pltpu.CompilerParams(needs_layout_passes=False),
        out_type=jax.ShapeDtypeStruct((E, cols), jnp.float32),
        scratch_types=[
            pltpu.VMEM((ch,), jnp.int32),
            pltpu.VMEM((ch, cols), jnp.float32),
            pltpu.SemaphoreType.DMA,
        ],
    )
    def k(table_hbm, idx_hbm, out_hbm, idx_v, rows_v, sem):
        wid = lax.axis_index("s") * _SC_CORES + lax.axis_index("c")
        base = wid * per

        def body(i, carry):
            off = base + i * ch
            pltpu.sync_copy(idx_hbm.at[pl.ds(off, ch)], idx_v)
            pltpu.async_copy(table_hbm.at[idx_v], rows_v, sem).wait()
            pltpu.sync_copy(rows_v, out_hbm.at[pl.ds(off, ch)])
            return carry

        lax.fori_loop(0, iters, body, 0)

    return k


@functools.lru_cache(maxsize=None)
def _scatter_call(cols):
    cpt = 8 if cols == 256 else 1
    ngrp = cols // cpt
    ch = 80
    iters = E // ch
    accw = N * cpt
    mesh = plsc.VectorSubcoreMesh(core_axis_name="c", subcore_axis_name="s")

    @functools.partial(
        pl.kernel,
        mesh=mesh,
        compiler_params=pltpu.CompilerParams(
            needs_layout_passes=False, use_tc_tiling_on_sc=False),
        out_type=jax.ShapeDtypeStruct((ngrp, accw), jnp.float32),
        scratch_types=[
            pltpu.VMEM((ch,), jnp.int32),
            pltpu.VMEM((ch,), jnp.int32),
            pltpu.VMEM((ch, cpt) if cpt > 1 else (ch,), jnp.float32),
            pltpu.VMEM((accw,), jnp.float32),
            pltpu.SemaphoreType.DMA,
        ],
    )
    def k(rows_hbm, idx_hbm, out_hbm, idx_v, gidx_v, stage_v, acc, sem):
        t = lax.axis_index("s") * _SC_CORES + lax.axis_index("c")

        @pl.when(t < ngrp)
        def _():
            def zbody(i, carry):
                acc[pl.ds(i * 16, 16)] = jnp.zeros((16,), jnp.float32)
                return carry

            lax.fori_loop(0, accw // 16, zbody, 0)

            lanes = lax.iota(jnp.int32, 16)

            def body(i, carry):
                off = i * ch
                pltpu.sync_copy(idx_hbm.at[pl.ds(off, ch)], idx_v)
                for g in range(ch // 16):
                    gidx_v[pl.ds(g * 16, 16)] = (
                        (off + g * 16 + lanes) * ngrp + t)
                pltpu.async_copy(rows_hbm.at[gidx_v], stage_v, sem).wait()
                for g in range(ch // 16):
                    dstv = idx_v[pl.ds(g * 16, 16)] * cpt
                    rowi = g * 16 + lanes
                    for j in range(cpt):
                        if cpt > 1:
                            vals = plsc.load_gather(
                                stage_v, [rowi, jnp.full((16,), j, jnp.int32)])
                        else:
                            vals = plsc.load_gather(stage_v, [rowi])
                        plsc.addupdate_scatter(acc, [dstv + j], vals)
                return carry

            lax.fori_loop(0, iters, body, 0)
            pltpu.sync_copy(acc, out_hbm.at[t])

    return k


def _gather_rows(table, idx):
    return _gather_call(table.shape[0], table.shape[1])(table, idx)


def _scatter_add(rows, idx):
    cols = rows.shape[1]
    cpt = 8 if cols == 256 else 1
    ngrp = cols // cpt
    rows_view = rows.reshape(E * ngrp, cpt)
    if cpt == 1:
        rows_view = rows_view.reshape(E * ngrp)
    out = _scatter_call(cols)(rows_view, idx)
    return out.reshape(ngrp, N, cpt).transpose(1, 0, 2).reshape(N, cols)


def _row(v):
    return v.reshape(1, -1)


def kernel(x, edge_index, edge_attr, batch, nA, nB, system_size, params):
    f32 = jnp.float32
    src = edge_index[0]
    dst = edge_index[1]

    xp = jnp.zeros((N, 128), f32).at[:, :4].set(x)
    ep_in = jnp.zeros((E, 128), f32).at[:, :3].set(edge_attr)
    pn = params['node_enc']
    wt = jnp.zeros((128, H), f32).at[:4, :].set(pn['W'].T)
    h = _enc_call(N, 128, NB)(xp, wt, _row(pn['b']), _row(pn['g']),
                              _row(pn['be']))
    pe = params['edge_enc']
    wt = jnp.zeros((128, H), f32).at[:3, :].set(pe['W'].T)
    e_enc = _enc_call(E, 128, EB)(ep_in, wt, _row(pe['b']), _row(pe['g']),
                                  _row(pe['be']))

    sel = (jnp.arange(H)[:, None] // C == jnp.arange(HEADS)[None, :])
    s_mat = sel.astype(f32)
    st_mat = s_mat.T
    sp16 = jnp.zeros((2 * HEADS, H), f32).at[:HEADS, :].set(st_mat)

    for i in range(NL):
        lp = params['layers'][i]
        if i % 2 == 0:
            hs = _gather_rows(h, src)
            e_enc, msg = _gine_edge_call()(
                e_enc, hs, lp['ec_W'].T, _row(lp['ec_b']), _row(lp['ec_g']),
                _row(lp['ec_be']))
            agg = _scatter_add(msg, dst)
            pp = params['pool'][i // 2]
            h = _gine_node_call()(
                h, agg, lp['W1'].T, _row(lp['b1']), _row(lp['g1']),
                _row(lp['be1']), lp['W2'].T, _row(lp['b2']),
                _row(lp['n_g']), _row(lp['n_be']),
                pp['W'].T, _row(pp['b']), _row(pp['g']), _row(pp['be']))
        else:
            e_enc, eproj = _tf_edge_call()(
                e_enc, lp['ec_W'].T, _row(lp['ec_b']), _row(lp['ec_g']),
                _row(lp['ec_be']), lp['We'].T)
            wcat = jnp.concatenate(
                [lp['Wq'].T, lp['Wk'].T, lp['Wv'].T, lp['Ws'].T], axis=1)
            bcat = jnp.concatenate(
                [lp['bq'], lp['bk'], lp['bv'], lp['bs']])
            qkvs = _qkvs_call()(h, wcat, _row(bcat))
            q = qkvs[:, 0:H]
            kk = qkvs[:, H:2 * H]
            v = qkvs[:, 2 * H:3 * H]
            xr = qkvs[:, 3 * H:4 * H]
            qd = _gather_rows(q, dst)
            ks = _gather_rows(kk, src)
            vs = _gather_rows(v, src)
            logits, m = _logits_call()(qd, ks, eproj, s_mat)
            exv, expad = _attnmsg_call()(logits, m, vs, eproj, st_mat)
            num = _scatter_add(exv, dst)
            den = _scatter_add(expad, dst)
            wbrep = jnp.broadcast_to(lp['Wb'].T, (3 * H, H))
            h = _tf_node_call()(num, den, xr, h, sp16, wbrep,
                                _row(lp['n_g']), _row(lp['n_be']))

    batchf = jnp.broadcast_to(batch.astype(f32)[:, None], (N, 128))
    oht = (jnp.arange(G)[:, None] == batch[None, :]).astype(f32)
    nap = jnp.broadcast_to(nA, (G, 128))
    nbp = jnp.broadcast_to(nB, (G, 128))
    ssp = jnp.broadcast_to(system_size, (G, 128))
    s0, s1 = params['s2s']
    pr = params['ro_proj']
    pg = params['gmlp']
    pf = params['final']
    wg1 = jnp.zeros((128, H), f32).at[:2, :].set(pg['W1'].T)
    wf3 = jnp.broadcast_to(pf['W3'].T, (128, 128))
    bf3 = jnp.broadcast_to(pf['b3'].reshape(1, 1), (1, 128))
    out = _s2s_tail_call()(
        h, batchf, oht, nap, nbp, ssp,
        s0['W_ih'].T, s0['W_hh'].T, _row(s0['b_ih']), _row(s0['b_hh']),
        s1['W_ih'].T, s1['W_hh'].T, _row(s1['b_ih']), _row(s1['b_hh']),
        pr['W'].T, _row(pr['b']), _row(pr['g']), _row(pr['be']),
        wg1, _row(pg['b1']), _row(pg['g']), _row(pg['be']),
        pg['W2'].T, _row(pg['b2']),
        pf['W1'].T, _row(pf['b1']), _row(pf['g1']), _row(pf['be1']),
        pf['W2'].T, _row(pf['b2']), _row(pf['g2']), _row(pf['be2']),
        wf3, bf3)
    return out[:, 0]


# speedup vs baseline: 4.7105x; 1.9545x over previous
import functools
import math

import jax
import jax.numpy as jnp
from jax import lax
from jax.experimental import pallas as pl
from jax.experimental.pallas import tpu as pltpu
from jax.experimental.pallas import tpu_sc as plsc

N = 10000
E = 160000
G = 16
H = 256
HEADS = 8
C = H // HEADS
NL = 8

EB = 2000
NB = 2000

_INTERPRET = False

_SC_CORES = 2
_SC_TILES = 16
_SC_WORKERS = _SC_CORES * _SC_TILES


def _rb(cols, blk):
    return pl.BlockSpec((blk, cols), lambda i: (i, 0))


def _wfull(shape):
    return pl.BlockSpec(shape, lambda i: (0,) * len(shape))


def _lnrow(y, g, be):
    m = jnp.mean(y, axis=1, keepdims=True)
    v = jnp.mean((y - m) ** 2, axis=1, keepdims=True)
    return (y - m) * lax.rsqrt(v + 1e-5) * g + be


def _silu(y):
    return y * jax.nn.sigmoid(y)


def _dot(a, b):
    return jnp.dot(a, b, preferred_element_type=jnp.float32)


def _enc_body(x_ref, wt_ref, b_ref, g_ref, be_ref, o_ref):
    y = _dot(x_ref[...], wt_ref[...]) + b_ref[...]
    o_ref[...] = _silu(_lnrow(y, g_ref[...], be_ref[...]))


@functools.lru_cache(maxsize=None)
def _enc_call(rows, kin, blk):
    return pl.pallas_call(
        _enc_body,
        grid=(rows // blk,),
        in_specs=[_rb(kin, blk), _wfull((kin, H)), _wfull((1, H)),
                  _wfull((1, H)), _wfull((1, H))],
        out_specs=_rb(H, blk),
        out_shape=jax.ShapeDtypeStruct((rows, H), jnp.float32),
        interpret=_INTERPRET,
    )


def _gine_edge_body(e_ref, hs_ref, wt_ref, b_ref, g_ref, be_ref,
                    eo_ref, msg_ref):
    y = _dot(e_ref[...], wt_ref[...]) + b_ref[...]
    e_new = _silu(_lnrow(y, g_ref[...], be_ref[...]))
    eo_ref[...] = e_new
    msg_ref[...] = jnp.maximum(hs_ref[...] + e_new, 0.0)


@functools.lru_cache(maxsize=None)
def _gine_edge_call():
    return pl.pallas_call(
        _gine_edge_body,
        grid=(E // EB,),
        in_specs=[_rb(H, EB), _rb(H, EB), _wfull((H, H)), _wfull((1, H)),
                  _wfull((1, H)), _wfull((1, H))],
        out_specs=[_rb(H, EB), _rb(H, EB)],
        out_shape=[jax.ShapeDtypeStruct((E, H), jnp.float32),
                   jax.ShapeDtypeStruct((E, H), jnp.float32)],
        interpret=_INTERPRET,
    )


def _gine_node_body(h_ref, agg_ref, w1t_ref, b1_ref, g1_ref, be1_ref,
                    w2t_ref, b2_ref, ng_ref, nbe_ref,
                    wpt_ref, bp_ref, gp_ref, bep_ref, o_ref):
    z = h_ref[...] + agg_ref[...]
    z = _dot(z, w1t_ref[...]) + b1_ref[...]
    z = _silu(_lnrow(z, g1_ref[...], be1_ref[...]))
    z = _dot(z, w2t_ref[...]) + b2_ref[...]
    h2 = h_ref[...] + _lnrow(z, ng_ref[...], nbe_ref[...])
    y = _dot(h2, wpt_ref[...]) + bp_ref[...]
    o_ref[...] = _silu(_lnrow(y, gp_ref[...], bep_ref[...]))


@functools.lru_cache(maxsize=None)
def _gine_node_call():
    return pl.pallas_call(
        _gine_node_body,
        grid=(N // NB,),
        in_specs=[_rb(H, NB), _rb(H, NB),
                  _wfull((H, H)), _wfull((1, H)), _wfull((1, H)),
                  _wfull((1, H)),
                  _wfull((H, H)), _wfull((1, H)),
                  _wfull((1, H)), _wfull((1, H)),
                  _wfull((H, H)), _wfull((1, H)), _wfull((1, H)),
                  _wfull((1, H))],
        out_specs=_rb(H, NB),
        out_shape=jax.ShapeDtypeStruct((N, H), jnp.float32),
        interpret=_INTERPRET,
    )


def _tf_edge_body(e_ref, wt_ref, b_ref, g_ref, be_ref, wet_ref,
                  eo_ref, ep_ref):
    y = _dot(e_ref[...], wt_ref[...]) + b_ref[...]
    e_new = _silu(_lnrow(y, g_ref[...], be_ref[...]))
    eo_ref[...] = e_new
    ep_ref[...] = _dot(e_new, wet_ref[...])


@functools.lru_cache(maxsize=None)
def _tf_edge_call():
    return pl.pallas_call(
        _tf_edge_body,
        grid=(E // EB,),
        in_specs=[_rb(H, EB), _wfull((H, H)), _wfull((1, H)),
                  _wfull((1, H)), _wfull((1, H)), _wfull((H, H))],
        out_specs=[_rb(H, EB), _rb(H, EB)],
        out_shape=[jax.ShapeDtypeStruct((E, H), jnp.float32),
                   jax.ShapeDtypeStruct((E, H), jnp.float32)],
        interpret=_INTERPRET,
    )


def _qkvs_body(h_ref, wt_ref, b_ref, o_ref):
    o_ref[...] = _dot(h_ref[...], wt_ref[...]) + b_ref[...]


@functools.lru_cache(maxsize=None)
def _qkvs_call():
    return pl.pallas_call(
        _qkvs_body,
        grid=(N // NB,),
        in_specs=[_rb(H, NB), _wfull((H, 4 * H)), _wfull((1, 4 * H))],
        out_specs=_rb(4 * H, NB),
        out_shape=jax.ShapeDtypeStruct((N, 4 * H), jnp.float32),
        interpret=_INTERPRET,
    )


def _logits_body(qd_ref, ks_ref, ep_ref, s_ref, lg_ref, m_ref):
    i = pl.program_id(0)
    kj = ks_ref[...] + ep_ref[...]
    prod = qd_ref[...] * kj
    lg = _dot(prod, s_ref[...]) * (1.0 / math.sqrt(float(C)))
    lg_ref[...] = lg
    mblk = jnp.max(lg, axis=0, keepdims=True)

    @pl.when(i == 0)
    def _():
        m_ref[...] = mblk

    @pl.when(i > 0)
    def _():
        m_ref[...] = jnp.maximum(m_ref[...], mblk)


@functools.lru_cache(maxsize=None)
def _logits_call():
    return pl.pallas_call(
        _logits_body,
        grid=(E // EB,),
        in_specs=[_rb(H, EB), _rb(H, EB), _rb(H, EB), _wfull((H, HEADS))],
        out_specs=[_rb(HEADS, EB), _wfull((1, HEADS))],
        out_shape=[jax.ShapeDtypeStruct((E, HEADS), jnp.float32),
                   jax.ShapeDtypeStruct((1, HEADS), jnp.float32)],
        interpret=_INTERPRET,
    )


def _attnmsg_body(lg_ref, m_ref, vs_ref, ep_ref, st_ref,
                  exv_ref, expad_ref):
    ex = jnp.exp(lg_ref[...] - m_ref[...])
    exb = _dot(ex, st_ref[...])
    exv_ref[...] = exb * (vs_ref[...] + ep_ref[...])
    expad_ref[...] = jnp.concatenate(
        [ex, jnp.zeros_like(ex)], axis=1)


@functools.lru_cache(maxsize=None)
def _attnmsg_call():
    return pl.pallas_call(
        _attnmsg_body,
        grid=(E // EB,),
        in_specs=[_rb(HEADS, EB), _wfull((1, HEADS)), _rb(H, EB),
                  _rb(H, EB), _wfull((HEADS, H))],
        out_specs=[_rb(H, EB), _rb(2 * HEADS, EB)],
        out_shape=[jax.ShapeDtypeStruct((E, H), jnp.float32),
                   jax.ShapeDtypeStruct((E, 2 * HEADS), jnp.float32)],
        interpret=_INTERPRET,
    )


def _tf_node_body(num_ref, den_ref, xr_ref, h_ref, sp_ref, wb_ref,
                  ng_ref, nbe_ref, o_ref):
    denb = _dot(den_ref[...], sp_ref[...])
    out = num_ref[...] / (denb + 1e-16)
    xr = xr_ref[...]
    cat = jnp.concatenate([out, xr, out - xr], axis=1)
    beta = jax.nn.sigmoid(_dot(cat, wb_ref[...]))
    h_new = beta * xr + (1.0 - beta) * out
    o_ref[...] = h_ref[...] + _lnrow(h_new, ng_ref[...], nbe_ref[...])


@functools.lru_cache(maxsize=None)
def _tf_node_call():
    return pl.pallas_call(
        _tf_node_body,
        grid=(N // NB,),
        in_specs=[_rb(H, NB), _rb(2 * HEADS, NB), _rb(H, NB), _rb(H, NB),
                  _wfull((2 * HEADS, H)), _wfull((3 * H, H)),
                  _wfull((1, H)), _wfull((1, H))],
        out_specs=_rb(H, NB),
        out_shape=jax.ShapeDtypeStruct((N, H), jnp.float32),
        interpret=_INTERPRET,
    )


def _s2s_tail_body(h_ref, bf_ref, oht_ref, nap_ref, nbp_ref, ssp_ref,
                   wih0_ref, whh0_ref, bih0_ref, bhh0_ref,
                   wih1_ref, whh1_ref, bih1_ref, bhh1_ref,
                   wro_ref, bro_ref, gro_ref, bero_ref,
                   wg1_ref, bg1_ref, gg_ref, beg_ref, wg2_ref, bg2_ref,
                   wf1_ref, bf1_ref, gf1_ref, bef1_ref,
                   wf2_ref, bf2_ref, gf2_ref, bef2_ref,
                   wf3_ref, bf3_ref, o_ref):
    hv = h_ref[...]
    bf = bf_ref[...]
    col = lax.broadcasted_iota(jnp.int32, (N, 128), 1).astype(jnp.float32)
    onehot = jnp.where(bf == col, 1.0, 0.0)
    oh16 = onehot[:, :G]
    oht = oht_ref[...]

    def s2s(wih, whh, bih, bhh):
        hs = jnp.zeros((G, H), jnp.float32)
        cs = jnp.zeros((G, H), jnp.float32)
        q_star = jnp.zeros((G, 2 * H), jnp.float32)
        for _ in range(4):
            gates = (_dot(q_star, wih) + bih + _dot(hs, whh) + bhh)
            ig = jax.nn.sigmoid(gates[:, 0:H])
            fg = jax.nn.sigmoid(gates[:, H:2 * H])
            gg_ = jnp.tanh(gates[:, 2 * H:3 * H])
            og = jax.nn.sigmoid(gates[:, 3 * H:4 * H])
            cs = fg * cs + ig * gg_
            hs = og * jnp.tanh(cs)
            qb = _dot(oh16, hs)
            e = jnp.sum(hv * qb, axis=1, keepdims=True)
            eb = e - 1e30 * (1.0 - onehot)
            m128 = jnp.max(eb, axis=0, keepdims=True)
            mb = jnp.sum(onehot * m128, axis=1, keepdims=True)
            ex = jnp.exp(e - mb)
            s128 = jnp.sum(onehot * ex, axis=0, keepdims=True)
            sb = jnp.sum(onehot * s128, axis=1, keepdims=True)
            a = ex / (sb + 1e-16)
            r = _dot(oht, a * hv)
            q_star = jnp.concatenate([hs, r], axis=1)
        return q_star

    q0 = s2s(wih0_ref[...], whh0_ref[...], bih0_ref[...], bhh0_ref[...])
    q1 = s2s(wih1_ref[...], whh1_ref[...], bih1_ref[...], bhh1_ref[...])
    h_r = jnp.concatenate([q0, q1], axis=1)
    h_r = _silu(_lnrow(_dot(h_r, wro_ref[...]) + bro_ref[...],
                       gro_ref[...], bero_ref[...]))
    ccol = lax.broadcasted_iota(jnp.int32, (G, 128), 1).astype(jnp.float32)
    nav = nap_ref[...] / (ssp_ref[...] + 1e-10)
    nbv = nbp_ref[...] / (ssp_ref[...] + 1e-10)
    gf = jnp.where(ccol == 0.0, nav, jnp.where(ccol == 1.0, nbv, 0.0))
    g1 = _silu(_lnrow(_dot(gf, wg1_ref[...]) + bg1_ref[...],
                      gg_ref[...], beg_ref[...]))
    gf_out = _dot(g1, wg2_ref[...]) + bg2_ref[...]
    comb = jnp.concatenate([h_r, gf_out], axis=1)
    z = _silu(_lnrow(_dot(comb, wf1_ref[...]) + bf1_ref[...],
                     gf1_ref[...], bef1_ref[...]))
    z = _silu(_lnrow(_dot(z, wf2_ref[...]) + bf2_ref[...],
                     gf2_ref[...], bef2_ref[...]))
    zz = _dot(z, wf3_ref[...]) + bf3_ref[...]
    o_ref[...] = jnp.maximum(zz, 0.0) + jnp.log(1.0 + jnp.exp(-jnp.abs(zz)))


@functools.lru_cache(maxsize=None)
def _s2s_tail_call():
    ws = [
        _wfull((N, H)), _wfull((N, 128)), _wfull((G, N)),
        _wfull((G, 128)), _wfull((G, 128)), _wfull((G, 128)),
        _wfull((2 * H, 4 * H)), _wfull((H, 4 * H)), _wfull((1, 4 * H)),
        _wfull((1, 4 * H)),
        _wfull((2 * H, 4 * H)), _wfull((H, 4 * H)), _wfull((1, 4 * H)),
        _wfull((1, 4 * H)),
        _wfull((4 * H, 2 * H)), _wfull((1, 2 * H)), _wfull((1, 2 * H)),
        _wfull((1, 2 * H)),
        _wfull((128, H)), _wfull((1, H)), _wfull((1, H)), _wfull((1, H)),
        _wfull((H, H)), _wfull((1, H)),
        _wfull((3 * H, H)), _wfull((1, H)), _wfull((1, H)), _wfull((1, H)),
        _wfull((H, 128)), _wfull((1, 128)), _wfull((1, 128)),
        _wfull((1, 128)),
        _wfull((128, 128)), _wfull((1, 128)),
    ]
    return pl.pallas_call(
        _s2s_tail_body,
        grid=(1,),
        in_specs=ws,
        out_specs=_wfull((G, 128)),
        out_shape=jax.ShapeDtypeStruct((G, 128), jnp.float32),
        interpret=_INTERPRET,
    )


@functools.lru_cache(maxsize=None)
def _gather_call(t_rows, cols):
    per = E // _SC_WORKERS
    ch = 200
    iters = per // ch
    mesh = plsc.VectorSubcoreMesh(core_axis_name="c", subcore_axis_name="s")

    @functools.partial(
        pl.kernel,
        mesh=mesh,
        compiler_params=pltpu.CompilerParams(needs_layout_passes=False),
        out_type=jax.ShapeDtypeStruct((E, cols), jnp.float32),
        scratch_types=[
            pltpu.VMEM((ch,), jnp.int32),
            pltpu.VMEM((ch, cols), jnp.float32),
            pltpu.SemaphoreType.DMA,
        ],
    )
    def k(table_hbm, idx_hbm, out_hbm, idx_v, rows_v, sem):
        wid = lax.axis_index("s") * _SC_CORES + lax.axis_index("c")
        base = wid * per

        def body(i, carry):
            off = base + i * ch
            pltpu.sync_copy(idx_hbm.at[pl.ds(off, ch)], idx_v)
            pltpu.async_copy(table_hbm.at[idx_v], rows_v, sem).wait()
            pltpu.sync_copy(rows_v, out_hbm.at[pl.ds(off, ch)])
            return carry

        lax.fori_loop(0, iters, body, 0)

    return k


@functools.lru_cache(maxsize=None)
def _scatter_call(cols):
    cpt = 8 if cols == 256 else 1
    ngrp = cols // cpt
    ch = 400
    iters = E // ch
    accw = N * cpt
    mesh = plsc.VectorSubcoreMesh(core_axis_name="c", subcore_axis_name="s")

    @functools.partial(
        pl.kernel,
        mesh=mesh,
        compiler_params=pltpu.CompilerParams(
            needs_layout_passes=False, use_tc_tiling_on_sc=False),
        out_type=jax.ShapeDtypeStruct((ngrp, accw), jnp.float32),
        scratch_types=[
            pltpu.VMEM((ch,), jnp.int32),
            pltpu.VMEM((ch,), jnp.int32),
            pltpu.VMEM((ch, cpt) if cpt > 1 else (ch,), jnp.float32),
            pltpu.VMEM((accw,), jnp.float32),
            pltpu.SemaphoreType.DMA,
        ],
    )
    def k(rows_hbm, idx_hbm, out_hbm, idx_v, gidx_v, stage_v, acc, sem):
        t = lax.axis_index("s") * _SC_CORES + lax.axis_index("c")

        @pl.when(t < ngrp)
        def _():
            def zbody(i, carry):
                acc[pl.ds(i * 16, 16)] = jnp.zeros((16,), jnp.float32)
                return carry

            lax.fori_loop(0, accw // 16, zbody, 0)

            lanes = lax.iota(jnp.int32, 16)

            def body(i, carry):
                off = i * ch
                pltpu.sync_copy(idx_hbm.at[pl.ds(off, ch)], idx_v)
                for g in range(ch // 16):
                    gidx_v[pl.ds(g * 16, 16)] = (
                        (off + g * 16 + lanes) * ngrp + t)
                pltpu.async_copy(rows_hbm.at[gidx_v], stage_v, sem).wait()
                for g in range(ch // 16):
                    dstv = idx_v[pl.ds(g * 16, 16)] * cpt
                    rowi = g * 16 + lanes
                    for j in range(cpt):
                        if cpt > 1:
                            vals = plsc.load_gather(
                                stage_v, [rowi, jnp.full((16,), j, jnp.int32)])
                        else:
                            vals = plsc.load_gather(stage_v, [rowi])
                        plsc.addupdate_scatter(acc, [dstv + j], vals)
                return carry

            lax.fori_loop(0, iters, body, 0)
            pltpu.sync_copy(acc, out_hbm.at[t])

    return k


def _gather_rows(table, idx):
    return _gather_call(table.shape[0], table.shape[1])(table, idx)


def _scatter_add(rows, idx):
    cols = rows.shape[1]
    cpt = 8 if cols == 256 else 1
    ngrp = cols // cpt
    rows_view = rows.reshape(E * ngrp, cpt)
    if cpt == 1:
        rows_view = rows_view.reshape(E * ngrp)
    out = _scatter_call(cols)(rows_view, idx)
    return out.reshape(ngrp, N, cpt).transpose(1, 0, 2).reshape(N, cols)


def _row(v):
    return v.reshape(1, -1)


def kernel(x, edge_index, edge_attr, batch, nA, nB, system_size, params):
    f32 = jnp.float32
    src = edge_index[0]
    dst = edge_index[1]

    xp = jnp.zeros((N, 128), f32).at[:, :4].set(x)
    ep_in = jnp.zeros((E, 128), f32).at[:, :3].set(edge_attr)
    pn = params['node_enc']
    wt = jnp.zeros((128, H), f32).at[:4, :].set(pn['W'].T)
    h = _enc_call(N, 128, NB)(xp, wt, _row(pn['b']), _row(pn['g']),
                              _row(pn['be']))
    pe = params['edge_enc']
    wt = jnp.zeros((128, H), f32).at[:3, :].set(pe['W'].T)
    e_enc = _enc_call(E, 128, EB)(ep_in, wt, _row(pe['b']), _row(pe['g']),
                                  _row(pe['be']))

    sel = (jnp.arange(H)[:, None] // C == jnp.arange(HEADS)[None, :])
    s_mat = sel.astype(f32)
    st_mat = s_mat.T
    sp16 = jnp.zeros((2 * HEADS, H), f32).at[:HEADS, :].set(st_mat)

    for i in range(NL):
        lp = params['layers'][i]
        if i % 2 == 0:
            hs = _gather_rows(h, src)
            e_enc, msg = _gine_edge_call()(
                e_enc, hs, lp['ec_W'].T, _row(lp['ec_b']), _row(lp['ec_g']),
                _row(lp['ec_be']))
            agg = _scatter_add(msg, dst)
            pp = params['pool'][i // 2]
            h = _gine_node_call()(
                h, agg, lp['W1'].T, _row(lp['b1']), _row(lp['g1']),
                _row(lp['be1']), lp['W2'].T, _row(lp['b2']),
                _row(lp['n_g']), _row(lp['n_be']),
                pp['W'].T, _row(pp['b']), _row(pp['g']), _row(pp['be']))
        else:
            e_enc, eproj = _tf_edge_call()(
                e_enc, lp['ec_W'].T, _row(lp['ec_b']), _row(lp['ec_g']),
                _row(lp['ec_be']), lp['We'].T)
            wcat = jnp.concatenate(
                [lp['Wq'].T, lp['Wk'].T, lp['Wv'].T, lp['Ws'].T], axis=1)
            bcat = jnp.concatenate(
                [lp['bq'], lp['bk'], lp['bv'], lp['bs']])
            qkvs = _qkvs_call()(h, wcat, _row(bcat))
            q = qkvs[:, 0:H]
            kk = qkvs[:, H:2 * H]
            v = qkvs[:, 2 * H:3 * H]
            xr = qkvs[:, 3 * H:4 * H]
            qd = _gather_rows(q, dst)
            ks = _gather_rows(kk, src)
            vs = _gather_rows(v, src)
            logits, m = _logits_call()(qd, ks, eproj, s_mat)
            exv, expad = _attnmsg_call()(logits, m, vs, eproj, st_mat)
            num = _scatter_add(exv, dst)
            den = _scatter_add(expad, dst)
            wbrep = jnp.broadcast_to(lp['Wb'].T, (3 * H, H))
            h = _tf_node_call()(num, den, xr, h, sp16, wbrep,
                                _row(lp['n_g']), _row(lp['n_be']))

    batchf = jnp.broadcast_to(batch.astype(f32)[:, None], (N, 128))
    oht = (jnp.arange(G)[:, None] == batch[None, :]).astype(f32)
    nap = jnp.broadcast_to(nA, (G, 128))
    nbp = jnp.broadcast_to(nB, (G, 128))
    ssp = jnp.broadcast_to(system_size, (G, 128))
    s0, s1 = params['s2s']
    pr = params['ro_proj']
    pg = params['gmlp']
    pf = params['final']
    wg1 = jnp.zeros((128, H), f32).at[:2, :].set(pg['W1'].T)
    wf3 = jnp.broadcast_to(pf['W3'].T, (128, 128))
    bf3 = jnp.broadcast_to(pf['b3'].reshape(1, 1), (1, 128))
    out = _s2s_tail_call()(
        h, batchf, oht, nap, nbp, ssp,
        s0['W_ih'].T, s0['W_hh'].T, _row(s0['b_ih']), _row(s0['b_hh']),
        s1['W_ih'].T, s1['W_hh'].T, _row(s1['b_ih']), _row(s1['b_hh']),
        pr['W'].T, _row(pr['b']), _row(pr['g']), _row(pr['be']),
        wg1, _row(pg['b1']), _row(pg['g']), _row(pg['be']),
        pg['W2'].T, _row(pg['b2']),
        pf['W1'].T, _row(pf['b1']), _row(pf['g1']), _row(pf['be1']),
        pf['W2'].T, _row(pf['b2']), _row(pf['g2']), _row(pf['be2']),
        wf3, bf3)
    return out[:, 0]


# speedup vs baseline: 6.9904x; 1.4840x over previous
import functools
import math

import jax
import jax.numpy as jnp
from jax import lax
from jax.experimental import pallas as pl
from jax.experimental.pallas import tpu as pltpu
from jax.experimental.pallas import tpu_sc as plsc

N = 10000
E = 160000
G = 16
H = 256
HEADS = 8
C = H // HEADS
NL = 8

EB = 2000
NB = 2000

_INTERPRET = False

_SC_CORES = 2
_SC_TILES = 16
_SC_WORKERS = _SC_CORES * _SC_TILES


def _rb(cols, blk):
    return pl.BlockSpec((blk, cols), lambda i: (i, 0))


def _wfull(shape):
    return pl.BlockSpec(shape, lambda i: (0,) * len(shape))


def _lnrow(y, g, be):
    m = jnp.mean(y, axis=1, keepdims=True)
    v = jnp.mean((y - m) ** 2, axis=1, keepdims=True)
    return (y - m) * lax.rsqrt(v + 1e-5) * g + be


def _silu(y):
    return y * jax.nn.sigmoid(y)


def _dot(a, b):
    return jnp.dot(a, b, preferred_element_type=jnp.float32)


def _enc_body(x_ref, wt_ref, b_ref, g_ref, be_ref, o_ref):
    y = _dot(x_ref[...], wt_ref[...]) + b_ref[...]
    o_ref[...] = _silu(_lnrow(y, g_ref[...], be_ref[...]))


@functools.lru_cache(maxsize=None)
def _enc_call(rows, kin, blk):
    return pl.pallas_call(
        _enc_body,
        grid=(rows // blk,),
        in_specs=[_rb(kin, blk), _wfull((kin, H)), _wfull((1, H)),
                  _wfull((1, H)), _wfull((1, H))],
        out_specs=_rb(H, blk),
        out_shape=jax.ShapeDtypeStruct((rows, H), jnp.float32),
        interpret=_INTERPRET,
    )


def _gine_edge_body(e_ref, hs_ref, wt_ref, b_ref, g_ref, be_ref,
                    eo_ref, msg_ref):
    y = _dot(e_ref[...], wt_ref[...]) + b_ref[...]
    e_new = _silu(_lnrow(y, g_ref[...], be_ref[...]))
    eo_ref[...] = e_new
    msg_ref[...] = jnp.maximum(hs_ref[...] + e_new, 0.0)


@functools.lru_cache(maxsize=None)
def _gine_edge_call():
    return pl.pallas_call(
        _gine_edge_body,
        grid=(E // EB,),
        in_specs=[_rb(H, EB), _rb(H, EB), _wfull((H, H)), _wfull((1, H)),
                  _wfull((1, H)), _wfull((1, H))],
        out_specs=[_rb(H, EB), _rb(H, EB)],
        out_shape=[jax.ShapeDtypeStruct((E, H), jnp.float32),
                   jax.ShapeDtypeStruct((E, H), jnp.float32)],
        interpret=_INTERPRET,
    )


def _gine_node_body(h_ref, agg_ref, w1t_ref, b1_ref, g1_ref, be1_ref,
                    w2t_ref, b2_ref, ng_ref, nbe_ref,
                    wpt_ref, bp_ref, gp_ref, bep_ref, o_ref):
    z = h_ref[...] + agg_ref[...]
    z = _dot(z, w1t_ref[...]) + b1_ref[...]
    z = _silu(_lnrow(z, g1_ref[...], be1_ref[...]))
    z = _dot(z, w2t_ref[...]) + b2_ref[...]
    h2 = h_ref[...] + _lnrow(z, ng_ref[...], nbe_ref[...])
    y = _dot(h2, wpt_ref[...]) + bp_ref[...]
    o_ref[...] = _silu(_lnrow(y, gp_ref[...], bep_ref[...]))


@functools.lru_cache(maxsize=None)
def _gine_node_call():
    return pl.pallas_call(
        _gine_node_body,
        grid=(N // NB,),
        in_specs=[_rb(H, NB), _rb(H, NB),
                  _wfull((H, H)), _wfull((1, H)), _wfull((1, H)),
                  _wfull((1, H)),
                  _wfull((H, H)), _wfull((1, H)),
                  _wfull((1, H)), _wfull((1, H)),
                  _wfull((H, H)), _wfull((1, H)), _wfull((1, H)),
                  _wfull((1, H))],
        out_specs=_rb(H, NB),
        out_shape=jax.ShapeDtypeStruct((N, H), jnp.float32),
        interpret=_INTERPRET,
    )


def _tf_edge_body(e_ref, wt_ref, b_ref, g_ref, be_ref, wet_ref,
                  eo_ref, ep_ref):
    y = _dot(e_ref[...], wt_ref[...]) + b_ref[...]
    e_new = _silu(_lnrow(y, g_ref[...], be_ref[...]))
    eo_ref[...] = e_new
    ep_ref[...] = _dot(e_new, wet_ref[...])


@functools.lru_cache(maxsize=None)
def _tf_edge_call():
    return pl.pallas_call(
        _tf_edge_body,
        grid=(E // EB,),
        in_specs=[_rb(H, EB), _wfull((H, H)), _wfull((1, H)),
                  _wfull((1, H)), _wfull((1, H)), _wfull((H, H))],
        out_specs=[_rb(H, EB), _rb(H, EB)],
        out_shape=[jax.ShapeDtypeStruct((E, H), jnp.float32),
                   jax.ShapeDtypeStruct((E, H), jnp.float32)],
        interpret=_INTERPRET,
    )


def _qkvs_body(h_ref, wt_ref, b_ref, o_ref):
    o_ref[...] = _dot(h_ref[...], wt_ref[...]) + b_ref[...]


@functools.lru_cache(maxsize=None)
def _qkvs_call():
    return pl.pallas_call(
        _qkvs_body,
        grid=(N // NB,),
        in_specs=[_rb(H, NB), _wfull((H, 4 * H)), _wfull((1, 4 * H))],
        out_specs=_rb(4 * H, NB),
        out_shape=jax.ShapeDtypeStruct((N, 4 * H), jnp.float32),
        interpret=_INTERPRET,
    )


def _logits_body(qd_ref, ks_ref, ep_ref, s_ref, lg_ref, m_ref):
    i = pl.program_id(0)
    kj = ks_ref[...] + ep_ref[...]
    prod = qd_ref[...] * kj
    lg = _dot(prod, s_ref[...]) * (1.0 / math.sqrt(float(C)))
    lg_ref[...] = lg
    mblk = jnp.max(lg, axis=0, keepdims=True)

    @pl.when(i == 0)
    def _():
        m_ref[...] = mblk

    @pl.when(i > 0)
    def _():
        m_ref[...] = jnp.maximum(m_ref[...], mblk)


@functools.lru_cache(maxsize=None)
def _logits_call():
    return pl.pallas_call(
        _logits_body,
        grid=(E // EB,),
        in_specs=[_rb(H, EB), _rb(H, EB), _rb(H, EB), _wfull((H, HEADS))],
        out_specs=[_rb(HEADS, EB), _wfull((1, HEADS))],
        out_shape=[jax.ShapeDtypeStruct((E, HEADS), jnp.float32),
                   jax.ShapeDtypeStruct((1, HEADS), jnp.float32)],
        interpret=_INTERPRET,
    )


def _attnmsg_body(lg_ref, m_ref, vs_ref, ep_ref, st_ref,
                  exv_ref, expad_ref):
    ex = jnp.exp(lg_ref[...] - m_ref[...])
    exb = _dot(ex, st_ref[...])
    exv_ref[...] = exb * (vs_ref[...] + ep_ref[...])
    expad_ref[...] = jnp.concatenate(
        [ex, jnp.zeros_like(ex)], axis=1)


@functools.lru_cache(maxsize=None)
def _attnmsg_call():
    return pl.pallas_call(
        _attnmsg_body,
        grid=(E // EB,),
        in_specs=[_rb(HEADS, EB), _wfull((1, HEADS)), _rb(H, EB),
                  _rb(H, EB), _wfull((HEADS, H))],
        out_specs=[_rb(H, EB), _rb(2 * HEADS, EB)],
        out_shape=[jax.ShapeDtypeStruct((E, H), jnp.float32),
                   jax.ShapeDtypeStruct((E, 2 * HEADS), jnp.float32)],
        interpret=_INTERPRET,
    )


def _tf_node_body(num_ref, den_ref, xr_ref, h_ref, sp_ref, wb_ref,
                  ng_ref, nbe_ref, o_ref):
    denb = _dot(den_ref[...], sp_ref[...])
    out = num_ref[...] / (denb + 1e-16)
    xr = xr_ref[...]
    cat = jnp.concatenate([out, xr, out - xr], axis=1)
    beta = jax.nn.sigmoid(_dot(cat, wb_ref[...]))
    h_new = beta * xr + (1.0 - beta) * out
    o_ref[...] = h_ref[...] + _lnrow(h_new, ng_ref[...], nbe_ref[...])


@functools.lru_cache(maxsize=None)
def _tf_node_call():
    return pl.pallas_call(
        _tf_node_body,
        grid=(N // NB,),
        in_specs=[_rb(H, NB), _rb(2 * HEADS, NB), _rb(H, NB), _rb(H, NB),
                  _wfull((2 * HEADS, H)), _wfull((3 * H, H)),
                  _wfull((1, H)), _wfull((1, H))],
        out_specs=_rb(H, NB),
        out_shape=jax.ShapeDtypeStruct((N, H), jnp.float32),
        interpret=_INTERPRET,
    )


def _s2s_tail_body(h_ref, bf_ref, oht_ref, nap_ref, nbp_ref, ssp_ref,
                   wih0_ref, whh0_ref, bih0_ref, bhh0_ref,
                   wih1_ref, whh1_ref, bih1_ref, bhh1_ref,
                   wro_ref, bro_ref, gro_ref, bero_ref,
                   wg1_ref, bg1_ref, gg_ref, beg_ref, wg2_ref, bg2_ref,
                   wf1_ref, bf1_ref, gf1_ref, bef1_ref,
                   wf2_ref, bf2_ref, gf2_ref, bef2_ref,
                   wf3_ref, bf3_ref, o_ref):
    hv = h_ref[...]
    bf = bf_ref[...]
    col = lax.broadcasted_iota(jnp.int32, (N, 128), 1).astype(jnp.float32)
    onehot = jnp.where(bf == col, 1.0, 0.0)
    oh16 = onehot[:, :G]
    oht = oht_ref[...]

    def s2s(wih, whh, bih, bhh):
        hs = jnp.zeros((G, H), jnp.float32)
        cs = jnp.zeros((G, H), jnp.float32)
        q_star = jnp.zeros((G, 2 * H), jnp.float32)
        for _ in range(4):
            gates = (_dot(q_star, wih) + bih + _dot(hs, whh) + bhh)
            ig = jax.nn.sigmoid(gates[:, 0:H])
            fg = jax.nn.sigmoid(gates[:, H:2 * H])
            gg_ = jnp.tanh(gates[:, 2 * H:3 * H])
            og = jax.nn.sigmoid(gates[:, 3 * H:4 * H])
            cs = fg * cs + ig * gg_
            hs = og * jnp.tanh(cs)
            qb = _dot(oh16, hs)
            e = jnp.sum(hv * qb, axis=1, keepdims=True)
            eb = e - 1e30 * (1.0 - onehot)
            m128 = jnp.max(eb, axis=0, keepdims=True)
            mb = jnp.sum(onehot * m128, axis=1, keepdims=True)
            ex = jnp.exp(e - mb)
            s128 = jnp.sum(onehot * ex, axis=0, keepdims=True)
            sb = jnp.sum(onehot * s128, axis=1, keepdims=True)
            a = ex / (sb + 1e-16)
            r = _dot(oht, a * hv)
            q_star = jnp.concatenate([hs, r], axis=1)
        return q_star

    q0 = s2s(wih0_ref[...], whh0_ref[...], bih0_ref[...], bhh0_ref[...])
    q1 = s2s(wih1_ref[...], whh1_ref[...], bih1_ref[...], bhh1_ref[...])
    h_r = jnp.concatenate([q0, q1], axis=1)
    h_r = _silu(_lnrow(_dot(h_r, wro_ref[...]) + bro_ref[...],
                       gro_ref[...], bero_ref[...]))
    ccol = lax.broadcasted_iota(jnp.int32, (G, 128), 1).astype(jnp.float32)
    nav = nap_ref[...] / (ssp_ref[...] + 1e-10)
    nbv = nbp_ref[...] / (ssp_ref[...] + 1e-10)
    gf = jnp.where(ccol == 0.0, nav, jnp.where(ccol == 1.0, nbv, 0.0))
    g1 = _silu(_lnrow(_dot(gf, wg1_ref[...]) + bg1_ref[...],
                      gg_ref[...], beg_ref[...]))
    gf_out = _dot(g1, wg2_ref[...]) + bg2_ref[...]
    comb = jnp.concatenate([h_r, gf_out], axis=1)
    z = _silu(_lnrow(_dot(comb, wf1_ref[...]) + bf1_ref[...],
                     gf1_ref[...], bef1_ref[...]))
    z = _silu(_lnrow(_dot(z, wf2_ref[...]) + bf2_ref[...],
                     gf2_ref[...], bef2_ref[...]))
    zz = _dot(z, wf3_ref[...]) + bf3_ref[...]
    o_ref[...] = jnp.maximum(zz, 0.0) + jnp.log(1.0 + jnp.exp(-jnp.abs(zz)))


@functools.lru_cache(maxsize=None)
def _s2s_tail_call():
    ws = [
        _wfull((N, H)), _wfull((N, 128)), _wfull((G, N)),
        _wfull((G, 128)), _wfull((G, 128)), _wfull((G, 128)),
        _wfull((2 * H, 4 * H)), _wfull((H, 4 * H)), _wfull((1, 4 * H)),
        _wfull((1, 4 * H)),
        _wfull((2 * H, 4 * H)), _wfull((H, 4 * H)), _wfull((1, 4 * H)),
        _wfull((1, 4 * H)),
        _wfull((4 * H, 2 * H)), _wfull((1, 2 * H)), _wfull((1, 2 * H)),
        _wfull((1, 2 * H)),
        _wfull((128, H)), _wfull((1, H)), _wfull((1, H)), _wfull((1, H)),
        _wfull((H, H)), _wfull((1, H)),
        _wfull((3 * H, H)), _wfull((1, H)), _wfull((1, H)), _wfull((1, H)),
        _wfull((H, 128)), _wfull((1, 128)), _wfull((1, 128)),
        _wfull((1, 128)),
        _wfull((128, 128)), _wfull((1, 128)),
    ]
    return pl.pallas_call(
        _s2s_tail_body,
        grid=(1,),
        in_specs=ws,
        out_specs=_wfull((G, 128)),
        out_shape=jax.ShapeDtypeStruct((G, 128), jnp.float32),
        interpret=_INTERPRET,
    )


@functools.lru_cache(maxsize=None)
def _gather_call(t_rows, cols):
    per = E // _SC_WORKERS
    ch = 200
    iters = per // ch
    mesh = plsc.VectorSubcoreMesh(core_axis_name="c", subcore_axis_name="s")

    @functools.partial(
        pl.kernel,
        mesh=mesh,
        compiler_params=pltpu.CompilerParams(needs_layout_passes=False),
        out_type=jax.ShapeDtypeStruct((E, cols), jnp.float32),
        scratch_types=[
            pltpu.VMEM((ch,), jnp.int32),
            pltpu.VMEM((ch,), jnp.int32),
            pltpu.VMEM((ch, cols), jnp.float32),
            pltpu.VMEM((ch, cols), jnp.float32),
            pltpu.SemaphoreType.DMA,
            pltpu.SemaphoreType.DMA,
            pltpu.SemaphoreType.DMA,
            pltpu.SemaphoreType.DMA,
            pltpu.SemaphoreType.DMA,
            pltpu.SemaphoreType.DMA,
        ],
    )
    def k(table_hbm, idx_hbm, out_hbm, i0, i1, r0, r1,
          si0, si1, sg0, sg1, so0, so1):
        wid = lax.axis_index("s") * _SC_CORES + lax.axis_index("c")
        base = wid * per
        idx_v = [i0, i1]
        rows_v = [r0, r1]
        si = [si0, si1]
        sg = [sg0, sg1]
        so = [so0, so1]

        def idx_copy(i):
            b = i % 2
            return pltpu.async_copy(
                idx_hbm.at[pl.ds(base + i * ch, ch)], idx_v[b], si[b])

        idx_cp = {}
        out_cp = {}
        idx_cp[0] = idx_copy(0)
        idx_cp[1] = idx_copy(1)
        for i in range(iters):
            b = i % 2
            idx_cp[i].wait()
            if i >= 2:
                out_cp[i - 2].wait()
            g = pltpu.async_copy(table_hbm.at[idx_v[b]], rows_v[b], sg[b])
            g.wait()
            if i + 2 < iters:
                idx_cp[i + 2] = idx_copy(i + 2)
            out_cp[i] = pltpu.async_copy(
                rows_v[b], out_hbm.at[pl.ds(base + i * ch, ch)], so[b])
        out_cp[iters - 2].wait()
        out_cp[iters - 1].wait()

    return k


@functools.lru_cache(maxsize=None)
def _scatter_call(cols):
    cpt = 8 if cols == 256 else 1
    ngrp = cols // cpt
    ch = 400
    iters = E // ch
    accw = N * cpt
    mesh = plsc.VectorSubcoreMesh(core_axis_name="c", subcore_axis_name="s")

    @functools.partial(
        pl.kernel,
        mesh=mesh,
        compiler_params=pltpu.CompilerParams(
            needs_layout_passes=False, use_tc_tiling_on_sc=False),
        out_type=jax.ShapeDtypeStruct((ngrp, accw), jnp.float32),
        scratch_types=[
            pltpu.VMEM((ch,), jnp.int32),
            pltpu.VMEM((ch,), jnp.int32),
            pltpu.VMEM((ch,), jnp.int32),
            pltpu.VMEM((ch,), jnp.int32),
            pltpu.VMEM((ch, cpt) if cpt > 1 else (ch,), jnp.float32),
            pltpu.VMEM((ch, cpt) if cpt > 1 else (ch,), jnp.float32),
            pltpu.VMEM((accw,), jnp.float32),
            pltpu.SemaphoreType.DMA,
            pltpu.SemaphoreType.DMA,
            pltpu.SemaphoreType.DMA,
            pltpu.SemaphoreType.DMA,
        ],
    )
    def k(rows_hbm, idx_hbm, out_hbm, i0, i1, g0, g1, s0, s1, acc,
          sem_i0, sem_i1, sem_s0, sem_s1):
        t = lax.axis_index("s") * _SC_CORES + lax.axis_index("c")

        @pl.when(t < ngrp)
        def _():
            def zbody(i, carry):
                acc[pl.ds(i * 16, 16)] = jnp.zeros((16,), jnp.float32)
                return carry

            lax.fori_loop(0, accw // 16, zbody, 0)

            lanes = lax.iota(jnp.int32, 16)
            idx_v = [i0, i1]
            gidx_v = [g0, g1]
            stage_v = [s0, s1]
            sem_i = [sem_i0, sem_i1]
            sem_s = [sem_s0, sem_s1]

            def issue(i, b):
                off = i * ch
                pltpu.async_copy(
                    idx_hbm.at[pl.ds(off, ch)], idx_v[b], sem_i[b])
                for g in range(ch // 16):
                    gidx_v[b][pl.ds(g * 16, 16)] = (
                        (off + g * 16 + lanes) * ngrp + t)
                pltpu.async_copy(rows_hbm.at[gidx_v[b]], stage_v[b],
                                 sem_s[b])

            def drain(b):
                pltpu.make_async_copy(
                    idx_hbm.at[pl.ds(0, ch)], idx_v[b], sem_i[b]).wait()
                pltpu.make_async_copy(
                    rows_hbm.at[gidx_v[b]], stage_v[b], sem_s[b]).wait()

            def accum(b):
                for g in range(ch // 16):
                    dstv = idx_v[b][pl.ds(g * 16, 16)] * cpt
                    rowi = g * 16 + lanes
                    for j in range(cpt):
                        if cpt > 1:
                            vals = plsc.load_gather(
                                stage_v[b],
                                [rowi, jnp.full((16,), j, jnp.int32)])
                        else:
                            vals = plsc.load_gather(stage_v[b], [rowi])
                        plsc.addupdate_scatter(acc, [dstv + j], vals)

            issue(0, 0)
            issue(1, 1)

            def body(k2, carry):
                for b in range(2):
                    i = k2 * 2 + b
                    drain(b)
                    accum(b)

                    @pl.when(i + 2 < iters)
                    def _():
                        issue(i + 2, b)

                return carry

            lax.fori_loop(0, iters // 2, body, 0)
            pltpu.sync_copy(acc, out_hbm.at[t])

    return k


def _gather_rows(table, idx):
    return _gather_call(table.shape[0], table.shape[1])(table, idx)


def _scatter_add(rows, idx):
    cols = rows.shape[1]
    cpt = 8 if cols == 256 else 1
    ngrp = cols // cpt
    rows_view = rows.reshape(E * ngrp, cpt)
    if cpt == 1:
        rows_view = rows_view.reshape(E * ngrp)
    out = _scatter_call(cols)(rows_view, idx)
    return out.reshape(ngrp, N, cpt).transpose(1, 0, 2).reshape(N, cols)


def _row(v):
    return v.reshape(1, -1)


def kernel(x, edge_index, edge_attr, batch, nA, nB, system_size, params):
    f32 = jnp.float32
    src = edge_index[0]
    dst = edge_index[1]

    xp = jnp.zeros((N, 128), f32).at[:, :4].set(x)
    ep_in = jnp.zeros((E, 128), f32).at[:, :3].set(edge_attr)
    pn = params['node_enc']
    wt = jnp.zeros((128, H), f32).at[:4, :].set(pn['W'].T)
    h = _enc_call(N, 128, NB)(xp, wt, _row(pn['b']), _row(pn['g']),
                              _row(pn['be']))
    pe = params['edge_enc']
    wt = jnp.zeros((128, H), f32).at[:3, :].set(pe['W'].T)
    e_enc = _enc_call(E, 128, EB)(ep_in, wt, _row(pe['b']), _row(pe['g']),
                                  _row(pe['be']))

    sel = (jnp.arange(H)[:, None] // C == jnp.arange(HEADS)[None, :])
    s_mat = sel.astype(f32)
    st_mat = s_mat.T
    sp16 = jnp.zeros((2 * HEADS, H), f32).at[:HEADS, :].set(st_mat)

    for i in range(NL):
        lp = params['layers'][i]
        if i % 2 == 0:
            hs = _gather_rows(h, src)
            e_enc, msg = _gine_edge_call()(
                e_enc, hs, lp['ec_W'].T, _row(lp['ec_b']), _row(lp['ec_g']),
                _row(lp['ec_be']))
            agg = _scatter_add(msg, dst)
            pp = params['pool'][i // 2]
            h = _gine_node_call()(
                h, agg, lp['W1'].T, _row(lp['b1']), _row(lp['g1']),
                _row(lp['be1']), lp['W2'].T, _row(lp['b2']),
                _row(lp['n_g']), _row(lp['n_be']),
                pp['W'].T, _row(pp['b']), _row(pp['g']), _row(pp['be']))
        else:
            e_enc, eproj = _tf_edge_call()(
                e_enc, lp['ec_W'].T, _row(lp['ec_b']), _row(lp['ec_g']),
                _row(lp['ec_be']), lp['We'].T)
            wcat = jnp.concatenate(
                [lp['Wq'].T, lp['Wk'].T, lp['Wv'].T, lp['Ws'].T], axis=1)
            bcat = jnp.concatenate(
                [lp['bq'], lp['bk'], lp['bv'], lp['bs']])
            qkvs = _qkvs_call()(h, wcat, _row(bcat))
            q = qkvs[:, 0:H]
            kk = qkvs[:, H:2 * H]
            v = qkvs[:, 2 * H:3 * H]
            xr = qkvs[:, 3 * H:4 * H]
            qd = _gather_rows(q, dst)
            ks = _gather_rows(kk, src)
            vs = _gather_rows(v, src)
            logits, m = _logits_call()(qd, ks, eproj, s_mat)
            exv, expad = _attnmsg_call()(logits, m, vs, eproj, st_mat)
            num = _scatter_add(exv, dst)
            den = _scatter_add(expad, dst)
            wbrep = jnp.broadcast_to(lp['Wb'].T, (3 * H, H))
            h = _tf_node_call()(num, den, xr, h, sp16, wbrep,
                                _row(lp['n_g']), _row(lp['n_be']))

    batchf = jnp.broadcast_to(batch.astype(f32)[:, None], (N, 128))
    oht = (jnp.arange(G)[:, None] == batch[None, :]).astype(f32)
    nap = jnp.broadcast_to(nA, (G, 128))
    nbp = jnp.broadcast_to(nB, (G, 128))
    ssp = jnp.broadcast_to(system_size, (G, 128))
    s0, s1 = params['s2s']
    pr = params['ro_proj']
    pg = params['gmlp']
    pf = params['final']
    wg1 = jnp.zeros((128, H), f32).at[:2, :].set(pg['W1'].T)
    wf3 = jnp.broadcast_to(pf['W3'].T, (128, 128))
    bf3 = jnp.broadcast_to(pf['b3'].reshape(1, 1), (1, 128))
    out = _s2s_tail_call()(
        h, batchf, oht, nap, nbp, ssp,
        s0['W_ih'].T, s0['W_hh'].T, _row(s0['b_ih']), _row(s0['b_hh']),
        s1['W_ih'].T, s1['W_hh'].T, _row(s1['b_ih']), _row(s1['b_hh']),
        pr['W'].T, _row(pr['b']), _row(pr['g']), _row(pr['be']),
        wg1, _row(pg['b1']), _row(pg['g']), _row(pg['be']),
        pg['W2'].T, _row(pg['b2']),
        pf['W1'].T, _row(pf['b1']), _row(pf['g1']), _row(pf['be1']),
        pf['W2'].T, _row(pf['b2']), _row(pf['g2']), _row(pf['be2']),
        wf3, bf3)
    return out[:, 0]
